# baseline (speedup 1.0000x reference)
import functools

import jax
import jax.numpy as jnp
from jax import lax
from jax.experimental import pallas as pl
from jax.experimental.pallas import tpu as pltpu
from jax.experimental.pallas import tpu_sc as plsc

NUM_HEADS = 9
HIST = 200
D = 32
NC = 2
NS = 16
NW = NC * NS
ROWS_PAD = 1856
RPW = 56
RPW_LAST = 64
NBUF = 16
LB = 128


def _sc_gather(item_t, items2d, user_t, uidx):
    mesh = plsc.VectorSubcoreMesh(
        core_axis_name="c", subcore_axis_name="s", num_cores=NC, num_subcores=NS
    )

    @functools.partial(
        pl.kernel,
        out_type=(
            jax.ShapeDtypeStruct((ROWS_PAD, D), jnp.float32),
            jax.ShapeDtypeStruct((8, D), jnp.float32),
        ),
        mesh=mesh,
        compiler_params=pltpu.CompilerParams(needs_layout_passes=False),
        scratch_types=[
            pltpu.VMEM((NUM_HEADS, HIST), jnp.int32),
            pltpu.VMEM((NBUF, D, LB), jnp.float32),
            pltpu.VMEM((RPW_LAST, D), jnp.float32),
            pltpu.VMEM((16,), jnp.int32),
            pltpu.VMEM((D, LB), jnp.float32),
            pltpu.VMEM((8, D), jnp.float32),
            pltpu.SemaphoreType.DMA,
            pltpu.SemaphoreType.DMA,
            pltpu.SemaphoreType.DMA,
        ],
    )
    def k(items_hbm, items2d_hbm, users_hbm, uidx_hbm, out_hbm, uout_hbm,
          items_v, blk_v, rows_v, uidx_v, ublk_v, urow_v, sem, usem, wsem):
        wid = lax.axis_index("s") * NC + lax.axis_index("c")
        base = wid * RPW
        pltpu.sync_copy(items2d_hbm, items_v)

        lane16 = lax.iota(jnp.int32, 16)
        chunks = []
        for t in range(RPW_LAST // 16):
            p = jnp.full((16,), base + t * 16, jnp.int32) + lane16
            hh = p // HIST
            cc = p - hh * HIST
            chunks.append(plsc.load_gather(items_v, [hh, cc]))
        idx_scalars = [chunks[r // 16][r % 16] for r in range(RPW_LAST)]

        def fire(r):
            i = idx_scalars[r]
            s = (i // LB) * LB
            return pltpu.async_copy(
                items_hbm.at[:, pl.ds(s, LB)], blk_v.at[r % NBUF], sem)

        def extract(r):
            i = idx_scalars[r]
            c = jnp.full((16,), i - (i // LB) * LB, jnp.int32)
            b = jnp.full((16,), r % NBUF, jnp.int32)
            for h in range(2):
                v = plsc.load_gather(blk_v, [b, lane16 + 16 * h, c])
                rows_v[r, pl.ds(16 * h, 16)] = v

        cps = [fire(r) for r in range(NBUF)]
        @pl.when(wid == 0)
        def _():
            pltpu.sync_copy(uidx_hbm, uidx_v)
            uchunk = uidx_v[pl.ds(0, 16)]
            ui = uchunk[0]
            us = (ui // LB) * LB
            pltpu.async_copy(
                users_hbm.at[:, pl.ds(us, LB)], ublk_v, usem).wait()
            uc = jnp.full((16,), ui - us, jnp.int32)
            for h in range(2):
                v = plsc.load_gather(ublk_v, [lane16 + 16 * h, uc])
                urow_v[0, pl.ds(16 * h, 16)] = v
            pltpu.sync_copy(urow_v, uout_hbm)

        wcps = []
        for r in range(RPW):
            cps[r % NBUF].wait()
            extract(r)
            nr = r + NBUF
            if nr < RPW:
                cps[nr % NBUF] = fire(nr)
            if r % 8 == 7:
                wcps.append(pltpu.async_copy(
                    rows_v.at[pl.ds(r - 7, 8)],
                    out_hbm.at[pl.ds(base + r - 7, 8)], wsem))

        @pl.when(wid == NW - 1)
        def _():
            tail = [fire(r) for r in range(RPW, RPW_LAST)]
            for j, r in enumerate(range(RPW, RPW_LAST)):
                tail[j].wait()
                extract(r)
            pltpu.sync_copy(rows_v.at[pl.ds(RPW, RPW_LAST - RPW)],
                            out_hbm.at[pl.ds(base + RPW, RPW_LAST - RPW)])

        for cp in wcps:
            cp.wait()

    return k(item_t, items2d, user_t, uidx)


def _tc_dense_body(gath_ref, urow_ref, w1_ref, b1_ref, w2_ref, b2_ref,
                   fw1_ref, fb1_ref, fw2_ref, fb2_ref, out_ref):
    ue_row = urow_ref[0:1, :]
    ue = jnp.broadcast_to(ue_row, (HIST, D))
    outs = []
    for i in range(NUM_HEADS):
        area = gath_ref[i * HIST:(i + 1) * HIST, :]
        x = jnp.concatenate([ue, area], axis=1)
        h = jnp.maximum(
            jnp.dot(x, w1_ref[i], preferred_element_type=jnp.float32)
            + b1_ref[i:i + 1, :], 0.0)
        o = (jnp.dot(h, w2_ref[i], preferred_element_type=jnp.float32)
             + b2_ref[i:i + 1, :])
        m = jnp.max(o, axis=0, keepdims=True)
        e = jnp.exp(o - m)
        s = jnp.sum(e, axis=0, keepdims=True)
        outs.append(jnp.sum((e / s) * area, axis=0, keepdims=True))
    area_all = jnp.concatenate(outs, axis=0)
    ue2 = jnp.broadcast_to(ue_row, (NUM_HEADS, D))
    uaa = jnp.concatenate([ue2, area_all], axis=1)
    h = jnp.maximum(
        jnp.dot(uaa, fw1_ref[...], preferred_element_type=jnp.float32)
        + fb1_ref[0:1, :], 0.0)
    o = (jnp.dot(h, fw2_ref[...], preferred_element_type=jnp.float32)
         + fb2_ref[0:1, :])
    m = jnp.max(o, axis=0, keepdims=True)
    e = jnp.exp(o - m)
    s = jnp.sum(e, axis=0, keepdims=True)
    out_ref[...] = jnp.sum((e / s) * area_all, axis=0, keepdims=True)


def kernel(user, input_items, U, I, W1, b1, W2, b2, fW1, fb1, fW2, fb2):
    uidx = jnp.full((16,), jnp.asarray(user, jnp.int32), jnp.int32)

    gath, urow = _sc_gather(I.T, input_items.astype(jnp.int32), U.T, uidx)

    out = pl.pallas_call(
        _tc_dense_body,
        out_shape=jax.ShapeDtypeStruct((1, D), jnp.float32),
    )(gath, urow, W1, b1, W2, b2, fW1, fb1.reshape(1, 16), fW2,
      fb2.reshape(1, 1))
    return out


# speedup vs baseline: 1.0306x; 1.0306x over previous
import functools

import jax
import jax.numpy as jnp
from jax import lax
from jax.experimental import pallas as pl
from jax.experimental.pallas import tpu as pltpu
from jax.experimental.pallas import tpu_sc as plsc

NUM_HEADS = 9
HIST = 200
D = 32
NC = 2
NS = 16
NW = NC * NS
ROWS_PAD = 1800
RPW = 56
RPW_LAST = 64
NBUF = 16
LB = 128


def _sc_gather(item_t, idx_flat, user_t, uidx):
    mesh = plsc.VectorSubcoreMesh(
        core_axis_name="c", subcore_axis_name="s", num_cores=NC, num_subcores=NS
    )

    @functools.partial(
        pl.kernel,
        out_type=(
            jax.ShapeDtypeStruct((ROWS_PAD, D), jnp.float32),
            jax.ShapeDtypeStruct((8, D), jnp.float32),
        ),
        mesh=mesh,
        compiler_params=pltpu.CompilerParams(needs_layout_passes=False),
        scratch_types=[
            pltpu.VMEM((RPW_LAST,), jnp.int32),
            pltpu.VMEM((NBUF, D, LB), jnp.float32),
            pltpu.VMEM((RPW_LAST, D), jnp.float32),
            pltpu.VMEM((16,), jnp.int32),
            pltpu.VMEM((D, LB), jnp.float32),
            pltpu.VMEM((8, D), jnp.float32),
            pltpu.SemaphoreType.DMA,
            pltpu.SemaphoreType.DMA,
            pltpu.SemaphoreType.DMA,
        ],
    )
    def k(items_hbm, idx_hbm, users_hbm, uidx_hbm, out_hbm, uout_hbm,
          idx_v, blk_v, rows_v, uidx_v, ublk_v, urow_v, sem, usem, wsem):
        wid = lax.axis_index("s") * NC + lax.axis_index("c")
        base = wid * RPW
        pltpu.sync_copy(idx_hbm.at[pl.ds(base, RPW_LAST)], idx_v)

        lane16 = lax.iota(jnp.int32, 16)
        chunks = [idx_v[pl.ds(t * 16, 16)] for t in range(RPW_LAST // 16)]
        idx_scalars = [chunks[r // 16][r % 16] for r in range(RPW_LAST)]

        def fire(r):
            i = idx_scalars[r]
            s = (i // LB) * LB
            return pltpu.async_copy(
                items_hbm.at[:, pl.ds(s, LB)], blk_v.at[r % NBUF], sem)

        def extract(r):
            i = idx_scalars[r]
            c = jnp.full((16,), i - (i // LB) * LB, jnp.int32)
            b = jnp.full((16,), r % NBUF, jnp.int32)
            for h in range(2):
                v = plsc.load_gather(blk_v, [b, lane16 + 16 * h, c])
                rows_v[r, pl.ds(16 * h, 16)] = v

        cps = [fire(r) for r in range(NBUF)]
        @pl.when(wid == 0)
        def _():
            pltpu.sync_copy(uidx_hbm, uidx_v)
            uchunk = uidx_v[pl.ds(0, 16)]
            ui = uchunk[0]
            us = (ui // LB) * LB
            pltpu.async_copy(
                users_hbm.at[:, pl.ds(us, LB)], ublk_v, usem).wait()
            uc = jnp.full((16,), ui - us, jnp.int32)
            for h in range(2):
                v = plsc.load_gather(ublk_v, [lane16 + 16 * h, uc])
                urow_v[0, pl.ds(16 * h, 16)] = v
            pltpu.sync_copy(urow_v, uout_hbm)

        for r in range(RPW):
            cps[r % NBUF].wait()
            extract(r)
            nr = r + NBUF
            if nr < RPW:
                cps[nr % NBUF] = fire(nr)
        pltpu.sync_copy(rows_v.at[pl.ds(0, RPW)], out_hbm.at[pl.ds(base, RPW)])

        @pl.when(wid == NW - 1)
        def _():
            tail = [fire(r) for r in range(RPW, RPW_LAST)]
            for j, r in enumerate(range(RPW, RPW_LAST)):
                tail[j].wait()
                extract(r)
            pltpu.sync_copy(rows_v.at[pl.ds(RPW, RPW_LAST - RPW)],
                            out_hbm.at[pl.ds(base + RPW, RPW_LAST - RPW)])

    return k(item_t, idx_flat, user_t, uidx)


def _tc_dense_body(gath_ref, urow_ref, w1_ref, b1_ref, w2_ref, b2_ref,
                   fw1_ref, fb1_ref, fw2_ref, fb2_ref, out_ref):
    ue_row = urow_ref[0:1, :]
    ue = jnp.broadcast_to(ue_row, (HIST, D))
    outs = []
    for i in range(NUM_HEADS):
        area = gath_ref[i * HIST:(i + 1) * HIST, :]
        x = jnp.concatenate([ue, area], axis=1)
        h = jnp.maximum(
            jnp.dot(x, w1_ref[i], preferred_element_type=jnp.float32)
            + b1_ref[i:i + 1, :], 0.0)
        o = (jnp.dot(h, w2_ref[i], preferred_element_type=jnp.float32)
             + b2_ref[i:i + 1, :])
        m = jnp.max(o, axis=0, keepdims=True)
        e = jnp.exp(o - m)
        s = jnp.sum(e, axis=0, keepdims=True)
        outs.append(jnp.sum((e / s) * area, axis=0, keepdims=True))
    area_all = jnp.concatenate(outs, axis=0)
    ue2 = jnp.broadcast_to(ue_row, (NUM_HEADS, D))
    uaa = jnp.concatenate([ue2, area_all], axis=1)
    h = jnp.maximum(
        jnp.dot(uaa, fw1_ref[...], preferred_element_type=jnp.float32)
        + fb1_ref[0:1, :], 0.0)
    o = (jnp.dot(h, fw2_ref[...], preferred_element_type=jnp.float32)
         + fb2_ref[0:1, :])
    m = jnp.max(o, axis=0, keepdims=True)
    e = jnp.exp(o - m)
    s = jnp.sum(e, axis=0, keepdims=True)
    out_ref[...] = jnp.sum((e / s) * area_all, axis=0, keepdims=True)


def kernel(user, input_items, U, I, W1, b1, W2, b2, fW1, fb1, fW2, fb2):
    uidx = jnp.full((16,), jnp.asarray(user, jnp.int32), jnp.int32)

    gath, urow = _sc_gather(I.T, input_items.reshape(-1).astype(jnp.int32),
                            U.T, uidx)

    out = pl.pallas_call(
        _tc_dense_body,
        out_shape=jax.ShapeDtypeStruct((1, D), jnp.float32),
    )(gath, urow, W1, b1, W2, b2, fW1, fb1.reshape(1, 16), fW2,
      fb2.reshape(1, 1))
    return out


# speedup vs baseline: 1.0395x; 1.0086x over previous
import functools

import jax
import jax.numpy as jnp
from jax import lax
from jax.experimental import pallas as pl
from jax.experimental.pallas import tpu as pltpu
from jax.experimental.pallas import tpu_sc as plsc

NUM_HEADS = 9
HIST = 200
D = 32
NC = 2
NS = 16
NW = NC * NS
ROWS_PAD = 1800
RPW = 56
RPW_LAST = 64
NBUF = 16
LB = 128


def _sc_gather(item_t, idx_flat, user_t, uidx):
    mesh = plsc.VectorSubcoreMesh(
        core_axis_name="c", subcore_axis_name="s", num_cores=NC, num_subcores=NS
    )

    @functools.partial(
        pl.kernel,
        out_type=(
            jax.ShapeDtypeStruct((ROWS_PAD, D), jnp.float32),
            jax.ShapeDtypeStruct((8, D), jnp.float32),
        ),
        mesh=mesh,
        compiler_params=pltpu.CompilerParams(needs_layout_passes=False),
        scratch_types=[
            pltpu.VMEM((RPW_LAST,), jnp.int32),
            pltpu.VMEM((NBUF, D, LB), jnp.float32),
            pltpu.VMEM((RPW_LAST, D), jnp.float32),
            pltpu.VMEM((16,), jnp.int32),
            pltpu.VMEM((D, LB), jnp.float32),
            pltpu.VMEM((8, D), jnp.float32),
            pltpu.SemaphoreType.DMA,
            pltpu.SemaphoreType.DMA,
            pltpu.SemaphoreType.DMA,
        ],
    )
    def k(items_hbm, idx_hbm, users_hbm, uidx_hbm, out_hbm, uout_hbm,
          idx_v, blk_v, rows_v, uidx_v, ublk_v, urow_v, sem, usem, wsem):
        wid = lax.axis_index("s") * NC + lax.axis_index("c")
        base = wid * RPW
        pltpu.sync_copy(idx_hbm.at[pl.ds(base, RPW_LAST)], idx_v)

        lane16 = lax.iota(jnp.int32, 16)
        chunks = [idx_v[pl.ds(t * 16, 16)] for t in range(RPW_LAST // 16)]
        idx_scalars = [chunks[r // 16][r % 16] for r in range(RPW_LAST)]

        def fire(r):
            i = idx_scalars[r]
            s = (i // LB) * LB
            return pltpu.async_copy(
                items_hbm.at[:, pl.ds(s, LB)], blk_v.at[r % NBUF], sem)

        def extract(r):
            i = idx_scalars[r]
            c = jnp.full((16,), i - (i // LB) * LB, jnp.int32)
            b = jnp.full((16,), r % NBUF, jnp.int32)
            for h in range(2):
                v = plsc.load_gather(blk_v, [b, lane16 + 16 * h, c])
                rows_v[r, pl.ds(16 * h, 16)] = v

        cps = [fire(r) for r in range(NBUF)]
        @pl.when(wid == NW - 2)
        def _():
            pltpu.sync_copy(uidx_hbm, uidx_v)
            uchunk = uidx_v[pl.ds(0, 16)]
            ui = uchunk[0]
            us = (ui // LB) * LB
            pltpu.async_copy(
                users_hbm.at[:, pl.ds(us, LB)], ublk_v, usem).wait()
            uc = jnp.full((16,), ui - us, jnp.int32)
            for h in range(2):
                v = plsc.load_gather(ublk_v, [lane16 + 16 * h, uc])
                urow_v[0, pl.ds(16 * h, 16)] = v
            pltpu.sync_copy(urow_v, uout_hbm)

        for r in range(RPW):
            cps[r % NBUF].wait()
            extract(r)
            nr = r + NBUF
            if nr < RPW:
                cps[nr % NBUF] = fire(nr)
        pltpu.sync_copy(rows_v.at[pl.ds(0, RPW)], out_hbm.at[pl.ds(base, RPW)])

        @pl.when(wid == NW - 1)
        def _():
            tail = [fire(r) for r in range(RPW, RPW_LAST)]
            for j, r in enumerate(range(RPW, RPW_LAST)):
                tail[j].wait()
                extract(r)
            pltpu.sync_copy(rows_v.at[pl.ds(RPW, RPW_LAST - RPW)],
                            out_hbm.at[pl.ds(base + RPW, RPW_LAST - RPW)])

    return k(item_t, idx_flat, user_t, uidx)


def _tc_dense_body(gath_ref, urow_ref, w1_ref, b1_ref, w2_ref, b2_ref,
                   fw1_ref, fb1_ref, fw2_ref, fb2_ref, out_ref):
    ue_row = urow_ref[0:1, :]
    ue = jnp.broadcast_to(ue_row, (HIST, D))
    outs = []
    for i in range(NUM_HEADS):
        area = gath_ref[i * HIST:(i + 1) * HIST, :]
        x = jnp.concatenate([ue, area], axis=1)
        h = jnp.maximum(
            jnp.dot(x, w1_ref[i], preferred_element_type=jnp.float32)
            + b1_ref[i:i + 1, :], 0.0)
        o = (jnp.dot(h, w2_ref[i], preferred_element_type=jnp.float32)
             + b2_ref[i:i + 1, :])
        m = jnp.max(o, axis=0, keepdims=True)
        e = jnp.exp(o - m)
        s = jnp.sum(e, axis=0, keepdims=True)
        outs.append(jnp.sum((e / s) * area, axis=0, keepdims=True))
    area_all = jnp.concatenate(outs, axis=0)
    ue2 = jnp.broadcast_to(ue_row, (NUM_HEADS, D))
    uaa = jnp.concatenate([ue2, area_all], axis=1)
    h = jnp.maximum(
        jnp.dot(uaa, fw1_ref[...], preferred_element_type=jnp.float32)
        + fb1_ref[0:1, :], 0.0)
    o = (jnp.dot(h, fw2_ref[...], preferred_element_type=jnp.float32)
         + fb2_ref[0:1, :])
    m = jnp.max(o, axis=0, keepdims=True)
    e = jnp.exp(o - m)
    s = jnp.sum(e, axis=0, keepdims=True)
    out_ref[...] = jnp.sum((e / s) * area_all, axis=0, keepdims=True)


def kernel(user, input_items, U, I, W1, b1, W2, b2, fW1, fb1, fW2, fb2):
    uidx = jnp.full((16,), jnp.asarray(user, jnp.int32), jnp.int32)

    gath, urow = _sc_gather(I.T, input_items.reshape(-1).astype(jnp.int32),
                            U.T, uidx)

    out = pl.pallas_call(
        _tc_dense_body,
        out_shape=jax.ShapeDtypeStruct((1, D), jnp.float32),
    )(gath, urow, W1, b1, W2, b2, fW1, fb1.reshape(1, 16), fW2,
      fb2.reshape(1, 1))
    return out
